# baseline (device time: 28396 ns/iter reference)
import os

import numpy as np
import jax
import jax.numpy as jnp
from jax import lax
from jax.experimental import pallas as pl
from jax.experimental.pallas import tpu as pltpu

N_DEV = 4
B, SQ, D = 2, 256, 768
DH = 64
HS = SQ // 2


def _rope_tables(hloc: int):
    inv = 1.0 / (10000.0 ** (np.arange(0, DH, 2) / DH))
    pos = np.arange(SQ)[:, None] * inv[None, :]
    cos = np.repeat(np.cos(pos), 2, axis=-1)
    sin = np.repeat(np.sin(pos), 2, axis=-1)
    cos_t = np.tile(cos, (B, hloc)).astype(np.float32)
    sin_t = np.tile(sin, (B, hloc)).astype(np.float32)
    r1 = np.zeros((DH, DH), np.float32)
    for k in range(DH // 2):
        r1[2 * k + 1, 2 * k] = -1.0
        r1[2 * k, 2 * k + 1] = 1.0
    r = np.kron(np.eye(hloc, dtype=np.float32), r1)
    return cos_t, sin_t, r


def kernel(x, Wq, Wk, Wv, Wo):
    hd = Wq.shape[1]
    hloc = hd // DH
    cos_np, sin_np, r_np = _rope_tables(hloc)
    cos_q = jnp.asarray(cos_np * 0.125, jnp.float32)
    sin_q = jnp.asarray(sin_np * 0.125, jnp.float32)
    cos_k = jnp.asarray(cos_np, jnp.float32)
    sin_k = jnp.asarray(sin_np, jnp.float32)
    r_c = jnp.asarray(r_np, jnp.bfloat16)
    x2 = x.reshape(B * SQ, D)
    skip_comm = bool(os.environ.get("KERNEL_SKIP_COMM"))

    def body(x_ref, wq_ref, wk_ref, wv_ref, wo_ref,
             cq_ref, sq_ref, ck_ref, sk_ref, r_ref,
             out_ref, comm_ref, send_sems, recv_sems):
        my = lax.axis_index("i")
        peer = [my ^ 1, 3 - my]

        barrier_sem = pltpu.get_barrier_semaphore()
        for p in range(2):
            pl.semaphore_signal(
                barrier_sem, inc=1,
                device_id=(peer[p],), device_id_type=pl.DeviceIdType.MESH,
            )
        pl.semaphore_wait(barrier_sem, 2)

        def slot(r, b, half, recv):
            return ((r * 2 + b) * 2 + half) * 2 + recv

        def exchange_start(r, b, half, data_bf16):
            dst = peer[half] if r == 0 else peer[1 - half]
            comm_ref[slot(r, b, half, 0)] = data_bf16
            rdma = pltpu.make_async_remote_copy(
                src_ref=comm_ref.at[slot(r, b, half, 0)],
                dst_ref=comm_ref.at[slot(r, b, half, 1)],
                send_sem=send_sems.at[r, b, half],
                recv_sem=recv_sems.at[r, b, half],
                device_id=(dst,),
                device_id_type=pl.DeviceIdType.MESH,
            )
            rdma.start()
            return rdma

        xb = x_ref[...].astype(jnp.bfloat16)
        wq = wq_ref[...].astype(jnp.bfloat16)
        wk = wk_ref[...].astype(jnp.bfloat16)
        wv = wv_ref[...].astype(jnp.bfloat16)
        wo = wo_ref[...].astype(jnp.bfloat16)
        rmat = r_ref[...]
        q = jnp.dot(xb, wq, preferred_element_type=jnp.float32)
        k = jnp.dot(xb, wk, preferred_element_type=jnp.float32)
        v = jnp.dot(xb, wv, preferred_element_type=jnp.float32).astype(
            jnp.bfloat16
        )
        q_rot = jnp.dot(q.astype(jnp.bfloat16), rmat,
                        preferred_element_type=jnp.float32)
        k_rot = jnp.dot(k.astype(jnp.bfloat16), rmat,
                        preferred_element_type=jnp.float32)
        qr = (q * cq_ref[...] + q_rot * sq_ref[...]).astype(jnp.bfloat16)
        kr = (k * ck_ref[...] + k_rot * sk_ref[...]).astype(jnp.bfloat16)

        def attn_partial(b):
            rows = slice(b * SQ, (b + 1) * SQ)
            ctxs = []
            for h in range(hloc):
                cols = slice(h * DH, (h + 1) * DH)
                s = lax.dot_general(
                    qr[rows, cols], kr[rows, cols],
                    (((1,), (1,)), ((), ())),
                    preferred_element_type=jnp.float32,
                )
                e = jnp.exp(s)
                r_inv = 1.0 / jnp.sum(e, axis=-1, keepdims=True)
                ctx = jnp.dot(e.astype(jnp.bfloat16), v[rows, cols],
                              preferred_element_type=jnp.float32)
                ctxs.append((ctx * r_inv).astype(jnp.bfloat16))
            ctx = jnp.concatenate(ctxs, axis=-1)
            return jnp.dot(ctx, wo, preferred_element_type=jnp.float32)

        if skip_comm:
            out_ref[0] = attn_partial(0)
            out_ref[1] = attn_partial(1)
            return

        parts, r0, acc, r1 = {}, {}, {}, {}

        def round0(b):
            p = attn_partial(b)
            parts[b] = p
            for half in range(2):
                r0[b, half] = exchange_start(
                    0, b, half,
                    p[half * HS:(half + 1) * HS].astype(jnp.bfloat16),
                )

        def round1(b):
            for half in range(2):
                r0[b, half].wait()
                acc[b, half] = (
                    parts[b][half * HS:(half + 1) * HS]
                    + comm_ref[slot(0, b, half, 1)].astype(jnp.float32)
                )
                r1[b, half] = exchange_start(
                    1, b, half, acc[b, half].astype(jnp.bfloat16)
                )

        def finish(b):
            for half in range(2):
                r1[b, half].wait()
                out_ref[b, half * HS:(half + 1) * HS] = (
                    acc[b, half]
                    + comm_ref[slot(1, b, half, 1)].astype(jnp.float32)
                )

        round0(0)
        round0(1)
        round1(0)
        round1(1)
        finish(0)
        finish(1)

    return pl.pallas_call(
        body,
        out_shape=jax.ShapeDtypeStruct((B, SQ, D), jnp.float32),
        in_specs=[pl.BlockSpec(memory_space=pltpu.VMEM)] * 10,
        out_specs=pl.BlockSpec(memory_space=pltpu.VMEM),
        scratch_shapes=[
            pltpu.VMEM((16, HS, D), jnp.bfloat16),
            pltpu.SemaphoreType.DMA((2, 2, 2)),
            pltpu.SemaphoreType.DMA((2, 2, 2)),
        ],
        compiler_params=pltpu.CompilerParams(collective_id=0),
    )(x2, Wq, Wk, Wv, Wo, cos_q, sin_q, cos_k, sin_k, r_c)


# device time: 22239 ns/iter; 1.2769x vs baseline; 1.2769x over previous
import os

import numpy as np
import jax
import jax.numpy as jnp
from jax import lax
from jax.experimental import pallas as pl
from jax.experimental.pallas import tpu as pltpu

N_DEV = 4
B, SQ, D = 2, 256, 768
DH = 64
HS = SQ // 2


def kernel(x, Wq, Wk, Wv, Wo):
    hd = Wq.shape[1]
    hloc = hd // DH
    skip_comm = bool(os.environ.get("KERNEL_SKIP_COMM"))

    def body(x_ref, wq_ref, wk_ref, wv_ref, wo_ref,
             out_ref, comm_ref, send_sems, recv_sems):
        my = lax.axis_index("i")
        peer = [my ^ 1, 3 - my]

        barrier_sem = pltpu.get_barrier_semaphore()
        for p in range(2):
            pl.semaphore_signal(
                barrier_sem, inc=1,
                device_id=(peer[p],), device_id_type=pl.DeviceIdType.MESH,
            )
        pl.semaphore_wait(barrier_sem, 2)

        def slot(r, b, half, recv):
            return ((r * 2 + b) * 2 + half) * 2 + recv

        def exchange_start(r, b, half, data_bf16):
            dst = peer[half] if r == 0 else peer[1 - half]
            comm_ref[slot(r, b, half, 0)] = data_bf16
            rdma = pltpu.make_async_remote_copy(
                src_ref=comm_ref.at[slot(r, b, half, 0)],
                dst_ref=comm_ref.at[slot(r, b, half, 1)],
                send_sem=send_sems.at[r, b, half],
                recv_sem=recv_sems.at[r, b, half],
                device_id=(dst,),
                device_id_type=pl.DeviceIdType.MESH,
            )
            rdma.start()
            return rdma

        lane = lax.broadcasted_iota(jnp.int32, (SQ, hd), 1)
        j2 = (lane % DH) // 2 * 2
        inv = jnp.exp(j2.astype(jnp.float32) * (-np.log(10000.0) / DH))
        posr = lax.broadcasted_iota(jnp.int32, (SQ, hd), 0)
        ang = posr.astype(jnp.float32) * inv
        cos_t = jnp.cos(ang)
        sin_t = jnp.sin(ang)
        even = (lane % 2) == 0

        def rope(t):
            rot = jnp.where(even, -jnp.roll(t, -1, axis=1),
                            jnp.roll(t, 1, axis=1))
            return t * cos_t + rot * sin_t

        wq = wq_ref[...].astype(jnp.bfloat16)
        wk = wk_ref[...].astype(jnp.bfloat16)
        wv = wv_ref[...].astype(jnp.bfloat16)
        wo = wo_ref[...].astype(jnp.bfloat16)

        def attn_partial(b):
            xb = x_ref[b].astype(jnp.bfloat16)
            q = jnp.dot(xb, wq, preferred_element_type=jnp.float32)
            k = jnp.dot(xb, wk, preferred_element_type=jnp.float32)
            v = jnp.dot(xb, wv, preferred_element_type=jnp.float32).astype(
                jnp.bfloat16
            )
            qr = (rope(q) * 0.125).astype(jnp.bfloat16)
            kr = rope(k).astype(jnp.bfloat16)
            ctxs = []
            for h in range(hloc):
                cols = slice(h * DH, (h + 1) * DH)
                s = lax.dot_general(
                    qr[:, cols], kr[:, cols],
                    (((1,), (1,)), ((), ())),
                    preferred_element_type=jnp.float32,
                )
                e = jnp.exp(s)
                r_inv = 1.0 / jnp.sum(e, axis=-1, keepdims=True)
                ctx = jnp.dot(e.astype(jnp.bfloat16), v[:, cols],
                              preferred_element_type=jnp.float32)
                ctxs.append((ctx * r_inv).astype(jnp.bfloat16))
            ctx = jnp.concatenate(ctxs, axis=-1)
            return jnp.dot(ctx, wo, preferred_element_type=jnp.float32)

        if skip_comm:
            out_ref[0] = attn_partial(0).astype(jnp.bfloat16)
            out_ref[1] = attn_partial(1).astype(jnp.bfloat16)
            return

        parts, r0, acc, r1 = {}, {}, {}, {}

        def round0(b):
            p = attn_partial(b)
            parts[b] = p
            for half in range(2):
                r0[b, half] = exchange_start(
                    0, b, half,
                    p[half * HS:(half + 1) * HS].astype(jnp.bfloat16),
                )

        def round1(b):
            for half in range(2):
                r0[b, half].wait()
                acc[b, half] = (
                    parts[b][half * HS:(half + 1) * HS]
                    + comm_ref[slot(0, b, half, 1)].astype(jnp.float32)
                )
                r1[b, half] = exchange_start(
                    1, b, half, acc[b, half].astype(jnp.bfloat16)
                )

        def finish(b):
            for half in range(2):
                r1[b, half].wait()
                out_ref[b, half * HS:(half + 1) * HS] = (
                    acc[b, half]
                    + comm_ref[slot(1, b, half, 1)].astype(jnp.float32)
                ).astype(jnp.bfloat16)

        round0(0)
        round0(1)
        round1(0)
        round1(1)
        finish(0)
        finish(1)

    return pl.pallas_call(
        body,
        out_shape=jax.ShapeDtypeStruct((B, SQ, D), jnp.bfloat16),
        in_specs=[pl.BlockSpec(memory_space=pltpu.VMEM)] * 5,
        out_specs=pl.BlockSpec(memory_space=pltpu.VMEM),
        scratch_shapes=[
            pltpu.VMEM((16, HS, D), jnp.bfloat16),
            pltpu.SemaphoreType.DMA((2, 2, 2)),
            pltpu.SemaphoreType.DMA((2, 2, 2)),
        ],
        compiler_params=pltpu.CompilerParams(collective_id=0),
    )(x, Wq, Wk, Wv, Wo)
